# baseline (speedup 1.0000x reference)
import functools
import math

import jax
import jax.numpy as jnp
from jax import lax
from jax.experimental import pallas as pl
from jax.experimental.pallas import tpu as pltpu
from jax.experimental.pallas import tpu_sc as plsc

H = 4
N = 1024
E = 2048
L = 8192
DN = 128
DE = 16

NT = 16
EN_T = E // NT
LN_T = L // NT
NROW_T = N // NT
EROW_T = E // NT
CH = 128

f32 = jnp.float32
i32 = jnp.int32


def _mlp_body(xn, xe,
              wnk1, bnk1, wnk2, bnk2, wnv1, bnv1, wnv2, bnv2,
              wek1, bek1, wek2, bek2, wev1, bev1, wev2, bev2,
              na_out, nv_out, ea_out, ev_out):
    def dot(a, b):
        return lax.dot_general(a, b, (((1,), (1,)), ((), ())),
                               preferred_element_type=f32)

    def relu(v):
        return jnp.maximum(v, 0.0)

    x_n = xn[...]
    x_e = xe[...]
    nk = relu(dot(relu(dot(x_n, wnk1[...]) + bnk1[...]), wnk2[...]) + bnk2[...])
    ek = relu(dot(relu(dot(x_e, wek1[...]) + bek1[...]), wek2[...]) + bek2[...])
    nv_out[...] = relu(dot(relu(dot(x_n, wnv1[...]) + bnv1[...]), wnv2[...]) + bnv2[...])
    ev_out[...] = relu(dot(relu(dot(x_e, wev1[...]) + bev1[...]), wev2[...]) + bev2[...])

    inv_sn = 1.0 / math.sqrt(DN)
    inv_se = 1.0 / math.sqrt(DE)
    bn = N // H
    be = E // H
    for h in range(H):
        blk = nk[h * bn:(h + 1) * bn, :] * inv_sn
        p = jnp.exp(blk - jnp.max(blk))
        na_out[h * bn:(h + 1) * bn, :] = p / jnp.sum(p)
        blk = ek[h * be:(h + 1) * be, :] * inv_se
        p = jnp.exp(blk - jnp.max(blk))
        ea_out[h * be:(h + 1) * be, :] = p / jnp.sum(p)


def _mlp_call(xn, xe, *weights):
    return pl.pallas_call(
        _mlp_body,
        out_shape=(
            jax.ShapeDtypeStruct((N, H), f32),
            jax.ShapeDtypeStruct((N, H * DN), f32),
            jax.ShapeDtypeStruct((E, H), f32),
            jax.ShapeDtypeStruct((E, H * DE), f32),
        ),
    )(xn, xe, *weights)


_GATHER_DNUMS = lax.GatherDimensionNumbers(
    offset_dims=(), collapsed_slice_dims=(0,), start_index_map=(0,))


def _splat(vec16, j):
    idx = jnp.full((16, 1), j, i32)
    return lax.gather(vec16, idx, _GATHER_DNUMS, slice_sizes=(1,),
                      mode=lax.GatherScatterMode.PROMISE_IN_BOUNDS)


def _win_body(gs, gd, ls, ld, tn_out, te_out, keyb, idb, ab, bb, lsb, ldb):
    cid = lax.axis_index("c")
    sid = lax.axis_index("s")

    def fill_and_scatter(a_ref, b_ref, table_ref, base, mult):
        pltpu.sync_copy(a_ref.at[pl.ds(base, CH)], ab)
        pltpu.sync_copy(b_ref.at[pl.ds(base, CH)], bb)

        def body(j, _):
            off = j * 16
            keyb[pl.ds(off, 16)] = ab[pl.ds(off, 16)] * mult + bb[pl.ds(off, 16)]
            idb[pl.ds(off, 16)] = lax.iota(i32, 16) + (base + off)
            return 0

        lax.fori_loop(0, CH // 16, body, 0)
        pltpu.sync_copy(idb, table_ref.at[keyb])

    @pl.when(cid == 0)
    def _():
        fill_and_scatter(gs, gd, tn_out, sid * EN_T, N)

    @pl.when(cid == 1)
    def _():
        base = sid * LN_T
        pltpu.sync_copy(ls.at[pl.ds(base, LN_T)], lsb)
        pltpu.sync_copy(ld.at[pl.ds(base, LN_T)], ldb)

        def chunk(c, _):
            cbase = c * CH
            def body(j, _):
                off = j * 16
                keyb[pl.ds(off, 16)] = (lsb[pl.ds(cbase + off, 16)] * E
                                        + ldb[pl.ds(cbase + off, 16)])
                idb[pl.ds(off, 16)] = lax.iota(i32, 16) + (base + cbase + off)
                return 0
            lax.fori_loop(0, CH // 16, body, 0)
            pltpu.sync_copy(idb, te_out.at[keyb])
            return 0
        lax.fori_loop(0, LN_T // CH, chunk, 0)


def _win_call(gs, gd, ls, ld):
    mesh = plsc.VectorSubcoreMesh(core_axis_name="c", subcore_axis_name="s")
    fn = pl.kernel(
        _win_body,
        out_type=(
            jax.ShapeDtypeStruct((N * N,), i32),
            jax.ShapeDtypeStruct((E * E,), i32),
        ),
        mesh=mesh,
        scratch_types=[
            pltpu.VMEM((CH,), i32),
            pltpu.VMEM((CH,), i32),
            pltpu.VMEM((CH,), i32),
            pltpu.VMEM((CH,), i32),
            pltpu.VMEM((LN_T,), i32),
            pltpu.VMEM((LN_T,), i32),
        ],
    )
    return fn(gs, gd, ls, ld)


def _agg_body(gs, gd, ls, ld, tn, te, naf, eaf, nv2, ev2,
              node_out, edge_out,
              s_node, s_edge,
              mbuf, bigbuf,
              srcb, dstb, keyb, idb, winb, wb, eab,
              idx4, sidx,
              lsb, ldb,
              navb, gdb, outb, sem):
    cid = lax.axis_index("c")
    sid = lax.axis_index("s")

    def zero_mbuf(nrows):
        def zrow(r, _):
            for k in range(128 // 16):
                mbuf[r, pl.ds(k * 16, 16)] = jnp.zeros((16,), f32)
            return 0
        lax.fori_loop(0, nrows, zrow, 0)

    @pl.when(cid == 0)
    def _():
        zero_mbuf(NROW_T)
        pltpu.sync_copy(mbuf.at[pl.ds(0, NROW_T)],
                        s_node.at[pl.ds(sid * NROW_T, NROW_T)])

        base = sid * EN_T
        pltpu.sync_copy(gs.at[pl.ds(base, EN_T)], srcb)
        pltpu.sync_copy(gd.at[pl.ds(base, EN_T)], dstb)
        for h in range(H):
            pltpu.sync_copy(eaf.at[pl.ds(h * E + base, EN_T)],
                            eab.at[pl.ds(h * EN_T, EN_T)])

        def kf(j, _):
            off = j * 16
            keyb[pl.ds(off, 16)] = (srcb[pl.ds(off, 16)] * N
                                    + dstb[pl.ds(off, 16)])
            return 0
        lax.fori_loop(0, EN_T // 16, kf, 0)
        plsc.subcore_barrier()
        pltpu.sync_copy(tn.at[keyb], winb)

        def wrow(j, _):
            off = j * 16
            ids = lax.iota(i32, 16) + (base + off)
            kept = jnp.where(winb[pl.ds(off, 16)] == ids, 1.0, 0.0).astype(f32)
            for h in range(H):
                wb[pl.ds(h * EN_T + off, 16)] = eab[pl.ds(h * EN_T + off, 16)] * kept
            return 0
        lax.fori_loop(0, EN_T // 16, wrow, 0)

        def sf(j, _):
            off = j * 16
            d16 = dstb[pl.ds(off, 16)]
            for h in range(H):
                idx4[h, pl.ds(off, 16)] = d16 + h * N
            return 0
        lax.fori_loop(0, EN_T // 16, sf, 0)
        for h in range(H):
            pltpu.sync_copy(nv2.at[idx4.at[h]],
                            bigbuf.at[pl.ds(h * EN_T, EN_T)])

        def msg(j, _):
            jhi = (j // 16) * 16
            jlo = j - jhi
            ws = [_splat(wb[pl.ds(h * EN_T + jhi, 16)], jlo) for h in range(H)]
            for k in range(DN // 16):
                acc = ws[0] * bigbuf[j, pl.ds(k * 16, 16)]
                for h in range(1, H):
                    acc = acc + ws[h] * bigbuf[h * EN_T + j, pl.ds(k * 16, 16)]
                mbuf[j, pl.ds(k * 16, 16)] = acc
            return 0
        lax.fori_loop(0, EN_T, msg, 0, unroll=4)

        def cpidx(j, _):
            off = j * 16
            sidx[pl.ds(off, 16)] = srcb[pl.ds(off, 16)]
            return 0
        lax.fori_loop(0, EN_T // 16, cpidx, 0)
        pltpu.sync_copy(mbuf, s_node.at[sidx], add=True)

        plsc.subcore_barrier()
        pltpu.sync_copy(s_node.at[pl.ds(sid * NROW_T, NROW_T)],
                        node_out.at[pl.ds(sid * NROW_T, NROW_T)])

    @pl.when(cid == 1)
    def _():
        zero_mbuf(EROW_T)
        pltpu.sync_copy(mbuf, s_edge.at[pl.ds(sid * EROW_T, EROW_T)])

        base = sid * LN_T
        pltpu.sync_copy(ls.at[pl.ds(base, LN_T)], lsb)
        pltpu.sync_copy(ld.at[pl.ds(base, LN_T)], ldb)
        plsc.subcore_barrier()

        def sub(c, _):
            cbase = c * CH
            def kf(j, _):
                off = j * 16
                keyb[pl.ds(off, 16)] = (lsb[pl.ds(cbase + off, 16)] * E
                                        + ldb[pl.ds(cbase + off, 16)])
                sidx[pl.ds(off, 16)] = lsb[pl.ds(cbase + off, 16)]
                return 0
            lax.fori_loop(0, CH // 16, kf, 0)
            pltpu.sync_copy(te.at[keyb], winb)

            def sf(j, _):
                off = j * 16
                ids = lax.iota(i32, 16) + (base + cbase + off)
                idx4[0, pl.ds(off, 16)] = jnp.where(
                    winb[pl.ds(off, 16)] == ids,
                    ldb[pl.ds(cbase + off, 16)],
                    jnp.full((16,), E, i32))
                return 0
            lax.fori_loop(0, CH // 16, sf, 0)
            pltpu.sync_copy(ev2.at[idx4.at[0]], mbuf)
            pltpu.sync_copy(mbuf, s_edge.at[sidx], add=True)
            return 0
        lax.fori_loop(0, LN_T // CH, sub, 0)

        plsc.subcore_barrier()

        rbase = sid * EROW_T
        pltpu.sync_copy(s_edge.at[pl.ds(rbase, EROW_T)],
                        bigbuf.at[pl.ds(CH, EROW_T)])
        pltpu.sync_copy(gd.at[pl.ds(rbase, EROW_T)], gdb)
        def shift(j2, _):
            g16 = gdb[pl.ds(j2 * 16, 16)]
            for h in range(H):
                idx4[h, pl.ds(j2 * 16, 16)] = g16 + h * N
            return 0
        lax.fori_loop(0, EROW_T // 16, shift, 0)
        for h in range(H):
            pltpu.sync_copy(naf.at[idx4.at[h]],
                            navb.at[pl.ds(h * EROW_T, EROW_T)])

        def orow(i, _):
            ihi = (i // 16) * 16
            ilo = i - ihi
            acc = jnp.zeros((16,), f32)
            for h in range(H):
                nas = _splat(navb[pl.ds(h * EROW_T + ihi, 16)], ilo)
                acc = acc + nas * bigbuf[CH + i, pl.ds(h * DE, 16)]
            outb[i, :] = acc
            return 0
        lax.fori_loop(0, EROW_T, orow, 0, unroll=4)
        pltpu.sync_copy(outb, edge_out.at[pl.ds(rbase, EROW_T)])


def _agg_scratch():
    return [
            pltpu.VMEM_SHARED((N, DN), f32),
            pltpu.VMEM_SHARED((E, 128), f32),
            pltpu.VMEM((128, 128), f32),
            pltpu.VMEM((H * EN_T, 128), f32),
            pltpu.VMEM((EN_T,), i32),
            pltpu.VMEM((EN_T,), i32),
            pltpu.VMEM((CH,), i32),
            pltpu.VMEM((CH,), i32),
            pltpu.VMEM((CH,), i32),
            pltpu.VMEM((H * EN_T,), f32),
            pltpu.VMEM((H * EN_T,), f32),
            pltpu.VMEM((H, CH), i32),
            pltpu.VMEM((CH,), i32),
            pltpu.VMEM((LN_T,), i32),
            pltpu.VMEM((LN_T,), i32),
            pltpu.VMEM((H * EROW_T,), f32),
            pltpu.VMEM((EROW_T,), i32),
            pltpu.VMEM((EROW_T, DE), f32),
            pltpu.SemaphoreType.DMA,
        ]


def _agg_call(gs, gd, ls, ld, tn, te, naf, eaf, nv2, ev2):
    mesh = plsc.VectorSubcoreMesh(core_axis_name="c", subcore_axis_name="s")
    fn = pl.kernel(
        _agg_body,
        out_type=(
            jax.ShapeDtypeStruct((N, DN), f32),
            jax.ShapeDtypeStruct((E, DE), f32),
        ),
        mesh=mesh,
        scratch_types=_agg_scratch(),
    )
    return fn(gs, gd, ls, ld, tn, te, naf, eaf, nv2, ev2)


def kernel(node_inputs, edge_inputs, g_edge_index, lg_edge_index,
           Wnk1, bnk1, Wnk2, bnk2, Wnv1, bnv1, Wnv2, bnv2,
           Wek1, bek1, Wek2, bek2, Wev1, bev1, Wev2, bev2):
    weights = [Wnk1, bnk1.reshape(1, -1), Wnk2, bnk2.reshape(1, -1),
               Wnv1, bnv1.reshape(1, -1), Wnv2, bnv2.reshape(1, -1),
               Wek1, bek1.reshape(1, -1), Wek2, bek2.reshape(1, -1),
               Wev1, bev1.reshape(1, -1), Wev2, bev2.reshape(1, -1)]
    na2d, nv, ea2d, ev = _mlp_call(node_inputs, edge_inputs, *weights)

    gs = g_edge_index[0].astype(i32)
    gd = g_edge_index[1].astype(i32)
    ls = lg_edge_index[0].astype(i32)
    ld = lg_edge_index[1].astype(i32)

    ev2 = ev.reshape(H * E, DE)
    evcat = jnp.concatenate(
        [ev2[h * E:(h + 1) * E] for h in range(H)]
        + [jnp.zeros((E, 128 - H * DE), f32)], axis=1)
    evcat = jnp.concatenate([evcat, jnp.zeros((8, 128), f32)], axis=0)
    tn, te = _win_call(gs, gd, ls, ld)
    node_out, edge_out = _agg_call(
        gs, gd, ls, ld, tn, te,
        na2d.reshape(-1), ea2d.reshape(-1),
        nv.reshape(H * N, DN), evcat)
    return node_out, edge_out


# speedup vs baseline: 1.0465x; 1.0465x over previous
import functools
import math

import jax
import jax.numpy as jnp
from jax import lax
from jax.experimental import pallas as pl
from jax.experimental.pallas import tpu as pltpu
from jax.experimental.pallas import tpu_sc as plsc

H = 4
N = 1024
E = 2048
L = 8192
DN = 128
DE = 16

NT = 16
EN_T = E // NT
LN_T = L // NT
NROW_T = N // NT
EROW_T = E // NT
CH = 128

f32 = jnp.float32
i32 = jnp.int32


def _mlp_body(xn, xe,
              wnk1, bnk1, wnk2, bnk2, wnv1, bnv1, wnv2, bnv2,
              wek1, bek1, wek2, bek2, wev1, bev1, wev2, bev2,
              na_out, nv_out, ea_out, ev_out):
    def dot(a, b):
        return lax.dot_general(a, b, (((1,), (1,)), ((), ())),
                               preferred_element_type=f32)

    def relu(v):
        return jnp.maximum(v, 0.0)

    x_n = xn[...]
    x_e = xe[...]
    nk = relu(dot(relu(dot(x_n, wnk1[...]) + bnk1[...]), wnk2[...]) + bnk2[...])
    ek = relu(dot(relu(dot(x_e, wek1[...]) + bek1[...]), wek2[...]) + bek2[...])
    nv_out[...] = relu(dot(relu(dot(x_n, wnv1[...]) + bnv1[...]), wnv2[...]) + bnv2[...])
    ev_out[...] = relu(dot(relu(dot(x_e, wev1[...]) + bev1[...]), wev2[...]) + bev2[...])

    inv_sn = 1.0 / math.sqrt(DN)
    inv_se = 1.0 / math.sqrt(DE)
    bn = N // H
    be = E // H
    for h in range(H):
        blk = nk[h * bn:(h + 1) * bn, :] * inv_sn
        p = jnp.exp(blk - jnp.max(blk))
        na_out[h * bn:(h + 1) * bn, :] = p / jnp.sum(p)
        blk = ek[h * be:(h + 1) * be, :] * inv_se
        p = jnp.exp(blk - jnp.max(blk))
        ea_out[h * be:(h + 1) * be, :] = p / jnp.sum(p)


def _mlp_call(xn, xe, *weights):
    return pl.pallas_call(
        _mlp_body,
        out_shape=(
            jax.ShapeDtypeStruct((N, H), f32),
            jax.ShapeDtypeStruct((N, H * DN), f32),
            jax.ShapeDtypeStruct((E, H), f32),
            jax.ShapeDtypeStruct((E, H * DE), f32),
        ),
    )(xn, xe, *weights)


_GATHER_DNUMS = lax.GatherDimensionNumbers(
    offset_dims=(), collapsed_slice_dims=(0,), start_index_map=(0,))


def _splat(vec16, j):
    idx = jnp.full((16, 1), j, i32)
    return lax.gather(vec16, idx, _GATHER_DNUMS, slice_sizes=(1,),
                      mode=lax.GatherScatterMode.PROMISE_IN_BOUNDS)


def _win_body(gs, gd, ls, ld, tn_out, te_out, keyb, idb, ab, bb, lsb, ldb):
    cid = lax.axis_index("c")
    sid = lax.axis_index("s")

    def fill_and_scatter(a_ref, b_ref, table_ref, base, mult):
        pltpu.sync_copy(a_ref.at[pl.ds(base, CH)], ab)
        pltpu.sync_copy(b_ref.at[pl.ds(base, CH)], bb)

        def body(j, _):
            off = j * 16
            keyb[pl.ds(off, 16)] = ab[pl.ds(off, 16)] * mult + bb[pl.ds(off, 16)]
            idb[pl.ds(off, 16)] = lax.iota(i32, 16) + (base + off)
            return 0

        lax.fori_loop(0, CH // 16, body, 0)
        pltpu.sync_copy(idb, table_ref.at[keyb])

    @pl.when(cid == 0)
    def _():
        fill_and_scatter(gs, gd, tn_out, sid * EN_T, N)

    wid = cid * NT + sid
    lw = L // (2 * NT)
    base = wid * lw
    pltpu.sync_copy(ls.at[pl.ds(base, lw)], lsb)
    pltpu.sync_copy(ld.at[pl.ds(base, lw)], ldb)

    def chunk(c, _):
        cbase = c * CH
        def body(j, _):
            off = j * 16
            keyb[pl.ds(off, 16)] = (lsb[pl.ds(cbase + off, 16)] * E
                                    + ldb[pl.ds(cbase + off, 16)])
            idb[pl.ds(off, 16)] = lax.iota(i32, 16) + (base + cbase + off)
            return 0
        lax.fori_loop(0, CH // 16, body, 0)
        pltpu.sync_copy(idb, te_out.at[keyb])
        return 0
    lax.fori_loop(0, lw // CH, chunk, 0)


def _win_call(gs, gd, ls, ld):
    mesh = plsc.VectorSubcoreMesh(core_axis_name="c", subcore_axis_name="s")
    fn = pl.kernel(
        _win_body,
        out_type=(
            jax.ShapeDtypeStruct((N * N,), i32),
            jax.ShapeDtypeStruct((E * E,), i32),
        ),
        mesh=mesh,
        scratch_types=[
            pltpu.VMEM((CH,), i32),
            pltpu.VMEM((CH,), i32),
            pltpu.VMEM((CH,), i32),
            pltpu.VMEM((CH,), i32),
            pltpu.VMEM((L // (2 * NT),), i32),
            pltpu.VMEM((L // (2 * NT),), i32),
        ],
    )
    return fn(gs, gd, ls, ld)


def _agg_body(gs, gd, ls, ld, tn, te, naf, eaf, nv2, ev2,
              node_out, edge_out,
              s_node, s_edge,
              mbuf, bigbuf,
              srcb, dstb, keyb, idb, winb, wb, eab,
              idx4, sidx,
              lsb, ldb,
              navb, nafb, gdb, outb, sem):
    cid = lax.axis_index("c")
    sid = lax.axis_index("s")

    def zero_mbuf(nrows):
        def zrow(r, _):
            for k in range(128 // 16):
                mbuf[r, pl.ds(k * 16, 16)] = jnp.zeros((16,), f32)
            return 0
        lax.fori_loop(0, nrows, zrow, 0)

    @pl.when(cid == 0)
    def _():
        zero_mbuf(NROW_T)
        pltpu.sync_copy(mbuf.at[pl.ds(0, NROW_T)],
                        s_node.at[pl.ds(sid * NROW_T, NROW_T)])

        base = sid * EN_T
        pltpu.sync_copy(gs.at[pl.ds(base, EN_T)], srcb)
        pltpu.sync_copy(gd.at[pl.ds(base, EN_T)], dstb)
        for h in range(H):
            pltpu.sync_copy(eaf.at[pl.ds(h * E + base, EN_T)],
                            eab.at[pl.ds(h * EN_T, EN_T)])

        def kf(j, _):
            off = j * 16
            keyb[pl.ds(off, 16)] = (srcb[pl.ds(off, 16)] * N
                                    + dstb[pl.ds(off, 16)])
            return 0
        lax.fori_loop(0, EN_T // 16, kf, 0)
        plsc.subcore_barrier()
        pltpu.sync_copy(tn.at[keyb], winb)

        def wrow(j, _):
            off = j * 16
            ids = lax.iota(i32, 16) + (base + off)
            kept = jnp.where(winb[pl.ds(off, 16)] == ids, 1.0, 0.0).astype(f32)
            for h in range(H):
                wb[pl.ds(h * EN_T + off, 16)] = eab[pl.ds(h * EN_T + off, 16)] * kept
            return 0
        lax.fori_loop(0, EN_T // 16, wrow, 0)

        def sf(j, _):
            off = j * 16
            d16 = dstb[pl.ds(off, 16)]
            for h in range(H):
                idx4[h, pl.ds(off, 16)] = d16 + h * N
            return 0
        lax.fori_loop(0, EN_T // 16, sf, 0)
        for h in range(H):
            pltpu.sync_copy(nv2.at[idx4.at[h]],
                            bigbuf.at[pl.ds(h * EN_T, EN_T)])

        def msg(j, _):
            jhi = (j // 16) * 16
            jlo = j - jhi
            ws = [_splat(wb[pl.ds(h * EN_T + jhi, 16)], jlo) for h in range(H)]
            for k in range(DN // 16):
                acc = ws[0] * bigbuf[j, pl.ds(k * 16, 16)]
                for h in range(1, H):
                    acc = acc + ws[h] * bigbuf[h * EN_T + j, pl.ds(k * 16, 16)]
                mbuf[j, pl.ds(k * 16, 16)] = acc
            return 0
        lax.fori_loop(0, EN_T, msg, 0, unroll=4)

        def cpidx(j, _):
            off = j * 16
            sidx[pl.ds(off, 16)] = srcb[pl.ds(off, 16)]
            return 0
        lax.fori_loop(0, EN_T // 16, cpidx, 0)
        pltpu.sync_copy(mbuf, s_node.at[sidx], add=True)

        plsc.subcore_barrier()
        pltpu.sync_copy(s_node.at[pl.ds(sid * NROW_T, NROW_T)],
                        node_out.at[pl.ds(sid * NROW_T, NROW_T)])

    @pl.when(cid == 1)
    def _():
        zero_mbuf(EROW_T)
        pltpu.sync_copy(mbuf, s_edge.at[pl.ds(sid * EROW_T, EROW_T)])

        base = sid * LN_T
        pltpu.sync_copy(ls.at[pl.ds(base, LN_T)], lsb)
        pltpu.sync_copy(ld.at[pl.ds(base, LN_T)], ldb)
        pltpu.sync_copy(naf, nafb)
        plsc.subcore_barrier()

        def sub(c, _):
            cbase = c * CH
            def kf(j, _):
                off = j * 16
                keyb[pl.ds(off, 16)] = (lsb[pl.ds(cbase + off, 16)] * E
                                        + ldb[pl.ds(cbase + off, 16)])
                sidx[pl.ds(off, 16)] = lsb[pl.ds(cbase + off, 16)]
                return 0
            lax.fori_loop(0, CH // 16, kf, 0)
            pltpu.sync_copy(te.at[keyb], winb)

            def sf(j, _):
                off = j * 16
                ids = lax.iota(i32, 16) + (base + cbase + off)
                idx4[0, pl.ds(off, 16)] = jnp.where(
                    winb[pl.ds(off, 16)] == ids,
                    ldb[pl.ds(cbase + off, 16)],
                    jnp.full((16,), E, i32))
                return 0
            lax.fori_loop(0, CH // 16, sf, 0)
            pltpu.sync_copy(ev2.at[idx4.at[0]], mbuf)
            pltpu.sync_copy(mbuf, s_edge.at[sidx], add=True)
            return 0
        lax.fori_loop(0, LN_T // CH, sub, 0)

        plsc.subcore_barrier()

        rbase = sid * EROW_T
        pltpu.sync_copy(s_edge.at[pl.ds(rbase, EROW_T)],
                        bigbuf.at[pl.ds(CH, EROW_T)])
        pltpu.sync_copy(gd.at[pl.ds(rbase, EROW_T)], gdb)
        def shift(j2, _):
            g16 = gdb[pl.ds(j2 * 16, 16)]
            for h in range(H):
                navb[pl.ds(h * EROW_T + j2 * 16, 16)] = plsc.load_gather(
                    nafb, [g16 + h * N])
            return 0
        lax.fori_loop(0, EROW_T // 16, shift, 0)

        def orow(i, _):
            ihi = (i // 16) * 16
            ilo = i - ihi
            acc = jnp.zeros((16,), f32)
            for h in range(H):
                nas = _splat(navb[pl.ds(h * EROW_T + ihi, 16)], ilo)
                acc = acc + nas * bigbuf[CH + i, pl.ds(h * DE, 16)]
            outb[i, :] = acc
            return 0
        lax.fori_loop(0, EROW_T, orow, 0, unroll=4)
        pltpu.sync_copy(outb, edge_out.at[pl.ds(rbase, EROW_T)])


def _agg_scratch():
    return [
            pltpu.VMEM_SHARED((N, DN), f32),
            pltpu.VMEM_SHARED((E, 128), f32),
            pltpu.VMEM((128, 128), f32),
            pltpu.VMEM((H * EN_T, 128), f32),
            pltpu.VMEM((EN_T,), i32),
            pltpu.VMEM((EN_T,), i32),
            pltpu.VMEM((CH,), i32),
            pltpu.VMEM((CH,), i32),
            pltpu.VMEM((CH,), i32),
            pltpu.VMEM((H * EN_T,), f32),
            pltpu.VMEM((H * EN_T,), f32),
            pltpu.VMEM((H, CH), i32),
            pltpu.VMEM((CH,), i32),
            pltpu.VMEM((LN_T,), i32),
            pltpu.VMEM((LN_T,), i32),
            pltpu.VMEM((H * EROW_T,), f32),
            pltpu.VMEM((H * N,), f32),
            pltpu.VMEM((EROW_T,), i32),
            pltpu.VMEM((EROW_T, DE), f32),
            pltpu.SemaphoreType.DMA,
        ]


def _agg_call(gs, gd, ls, ld, tn, te, naf, eaf, nv2, ev2):
    mesh = plsc.VectorSubcoreMesh(core_axis_name="c", subcore_axis_name="s")
    fn = pl.kernel(
        _agg_body,
        out_type=(
            jax.ShapeDtypeStruct((N, DN), f32),
            jax.ShapeDtypeStruct((E, DE), f32),
        ),
        mesh=mesh,
        scratch_types=_agg_scratch(),
        compiler_params=pltpu.CompilerParams(needs_layout_passes=False),
    )
    return fn(gs, gd, ls, ld, tn, te, naf, eaf, nv2, ev2)


def kernel(node_inputs, edge_inputs, g_edge_index, lg_edge_index,
           Wnk1, bnk1, Wnk2, bnk2, Wnv1, bnv1, Wnv2, bnv2,
           Wek1, bek1, Wek2, bek2, Wev1, bev1, Wev2, bev2):
    weights = [Wnk1, bnk1.reshape(1, -1), Wnk2, bnk2.reshape(1, -1),
               Wnv1, bnv1.reshape(1, -1), Wnv2, bnv2.reshape(1, -1),
               Wek1, bek1.reshape(1, -1), Wek2, bek2.reshape(1, -1),
               Wev1, bev1.reshape(1, -1), Wev2, bev2.reshape(1, -1)]
    na2d, nv, ea2d, ev = _mlp_call(node_inputs, edge_inputs, *weights)

    gs = g_edge_index[0].astype(i32)
    gd = g_edge_index[1].astype(i32)
    ls = lg_edge_index[0].astype(i32)
    ld = lg_edge_index[1].astype(i32)

    ev2 = ev.reshape(H * E, DE)
    evcat = jnp.concatenate(
        [ev2[h * E:(h + 1) * E] for h in range(H)]
        + [jnp.zeros((E, 128 - H * DE), f32)], axis=1)
    evcat = jnp.concatenate([evcat, jnp.zeros((8, 128), f32)], axis=0)
    tn, te = _win_call(gs, gd, ls, ld)
    node_out, edge_out = _agg_call(
        gs, gd, ls, ld, tn, te,
        na2d.reshape(-1), ea2d.reshape(-1),
        nv.reshape(H * N, DN), evcat)
    return node_out, edge_out


# speedup vs baseline: 1.0883x; 1.0399x over previous
import functools
import math

import jax
import jax.numpy as jnp
from jax import lax
from jax.experimental import pallas as pl
from jax.experimental.pallas import tpu as pltpu
from jax.experimental.pallas import tpu_sc as plsc

H = 4
N = 1024
E = 2048
L = 8192
DN = 128
DE = 16

NT = 16
EN_T = E // NT
LN_T = L // NT
NROW_T = N // NT
EROW_T = E // NT
CH = 128

f32 = jnp.float32
i32 = jnp.int32


def _mlp_body(xn, xe,
              wnk1, bnk1, wnk2, bnk2, wnv1, bnv1, wnv2, bnv2,
              wek1, bek1, wek2, bek2, wev1, bev1, wev2, bev2,
              na_out, nv_out, ea_out, ev_out):
    def dot(a, b):
        return lax.dot_general(a, b, (((1,), (1,)), ((), ())),
                               preferred_element_type=f32)

    def relu(v):
        return jnp.maximum(v, 0.0)

    x_n = xn[...]
    x_e = xe[...]
    nk = relu(dot(relu(dot(x_n, wnk1[...]) + bnk1[...]), wnk2[...]) + bnk2[...])
    ek = relu(dot(relu(dot(x_e, wek1[...]) + bek1[...]), wek2[...]) + bek2[...])
    nv_out[...] = relu(dot(relu(dot(x_n, wnv1[...]) + bnv1[...]), wnv2[...]) + bnv2[...])
    ev_out[...] = relu(dot(relu(dot(x_e, wev1[...]) + bev1[...]), wev2[...]) + bev2[...])

    inv_sn = 1.0 / math.sqrt(DN)
    inv_se = 1.0 / math.sqrt(DE)
    bn = N // H
    be = E // H
    for h in range(H):
        blk = nk[h * bn:(h + 1) * bn, :] * inv_sn
        p = jnp.exp(blk - jnp.max(blk))
        na_out[h * bn:(h + 1) * bn, :] = p / jnp.sum(p)
        blk = ek[h * be:(h + 1) * be, :] * inv_se
        p = jnp.exp(blk - jnp.max(blk))
        ea_out[h * be:(h + 1) * be, :] = p / jnp.sum(p)


def _mlp_call(xn, xe, *weights):
    return pl.pallas_call(
        _mlp_body,
        out_shape=(
            jax.ShapeDtypeStruct((N, H), f32),
            jax.ShapeDtypeStruct((N, H * DN), f32),
            jax.ShapeDtypeStruct((E, H), f32),
            jax.ShapeDtypeStruct((E, H * DE), f32),
        ),
    )(xn, xe, *weights)


_GATHER_DNUMS = lax.GatherDimensionNumbers(
    offset_dims=(), collapsed_slice_dims=(0,), start_index_map=(0,))


def _splat(vec16, j):
    idx = jnp.full((16, 1), j, i32)
    return lax.gather(vec16, idx, _GATHER_DNUMS, slice_sizes=(1,),
                      mode=lax.GatherScatterMode.PROMISE_IN_BOUNDS)


def _win_body(gs, gd, ls, ld, tn_out, te_out, keyb, idb, ab, bb, lsb, ldb):
    cid = lax.axis_index("c")
    sid = lax.axis_index("s")

    def fill_and_scatter(a_ref, b_ref, table_ref, base, mult):
        pltpu.sync_copy(a_ref.at[pl.ds(base, CH)], ab)
        pltpu.sync_copy(b_ref.at[pl.ds(base, CH)], bb)

        def body(j, _):
            off = j * 16
            keyb[pl.ds(off, 16)] = ab[pl.ds(off, 16)] * mult + bb[pl.ds(off, 16)]
            idb[pl.ds(off, 16)] = lax.iota(i32, 16) + (base + off)
            return 0

        lax.fori_loop(0, CH // 16, body, 0)
        pltpu.sync_copy(idb, table_ref.at[keyb])

    @pl.when(cid == 0)
    def _():
        fill_and_scatter(gs, gd, tn_out, sid * EN_T, N)

    wid = cid * NT + sid
    lw = L // (2 * NT)
    base = wid * lw
    pltpu.sync_copy(ls.at[pl.ds(base, lw)], lsb)
    pltpu.sync_copy(ld.at[pl.ds(base, lw)], ldb)

    def chunk(c, _):
        cbase = c * CH
        def body(j, _):
            off = j * 16
            keyb[pl.ds(off, 16)] = (lsb[pl.ds(cbase + off, 16)] * E
                                    + ldb[pl.ds(cbase + off, 16)])
            idb[pl.ds(off, 16)] = lax.iota(i32, 16) + (base + cbase + off)
            return 0
        lax.fori_loop(0, CH // 16, body, 0)
        pltpu.sync_copy(idb, te_out.at[keyb])
        return 0
    lax.fori_loop(0, lw // CH, chunk, 0)


def _win_call(gs, gd, ls, ld):
    mesh = plsc.VectorSubcoreMesh(core_axis_name="c", subcore_axis_name="s")
    fn = pl.kernel(
        _win_body,
        out_type=(
            jax.ShapeDtypeStruct((N * N,), i32),
            jax.ShapeDtypeStruct((E * E,), i32),
        ),
        mesh=mesh,
        scratch_types=[
            pltpu.VMEM((CH,), i32),
            pltpu.VMEM((CH,), i32),
            pltpu.VMEM((CH,), i32),
            pltpu.VMEM((CH,), i32),
            pltpu.VMEM((L // (2 * NT),), i32),
            pltpu.VMEM((L // (2 * NT),), i32),
        ],
    )
    return fn(gs, gd, ls, ld)


def _agg_body(gs, gd, ls, ld, tn, te, naf, eaf, nv2, ev2,
              node_out, edge_out,
              s_node, s_edge,
              mbuf, bigbuf,
              srcb, dstb, keyb, idb, winb, wb, eab,
              idx4, sidx,
              lsb, ldb, keyb4, winb4, idx44, sidx4,
              navb, nafb, gdb, outb, sem):
    cid = lax.axis_index("c")
    sid = lax.axis_index("s")

    def zero_mbuf(nrows):
        def zrow(r, _):
            for k in range(128 // 16):
                mbuf[r, pl.ds(k * 16, 16)] = jnp.zeros((16,), f32)
            return 0
        lax.fori_loop(0, nrows, zrow, 0)

    @pl.when(cid == 0)
    def _():
        zero_mbuf(NROW_T)
        pltpu.sync_copy(mbuf.at[pl.ds(0, NROW_T)],
                        s_node.at[pl.ds(sid * NROW_T, NROW_T)])

        base = sid * EN_T
        pltpu.sync_copy(gs.at[pl.ds(base, EN_T)], srcb)
        pltpu.sync_copy(gd.at[pl.ds(base, EN_T)], dstb)
        for h in range(H):
            pltpu.sync_copy(eaf.at[pl.ds(h * E + base, EN_T)],
                            eab.at[pl.ds(h * EN_T, EN_T)])

        def kf(j, _):
            off = j * 16
            keyb[pl.ds(off, 16)] = (srcb[pl.ds(off, 16)] * N
                                    + dstb[pl.ds(off, 16)])
            return 0
        lax.fori_loop(0, EN_T // 16, kf, 0)
        plsc.subcore_barrier()
        pltpu.sync_copy(tn.at[keyb], winb)

        def wrow(j, _):
            off = j * 16
            ids = lax.iota(i32, 16) + (base + off)
            kept = jnp.where(winb[pl.ds(off, 16)] == ids, 1.0, 0.0).astype(f32)
            for h in range(H):
                wb[pl.ds(h * EN_T + off, 16)] = eab[pl.ds(h * EN_T + off, 16)] * kept
            return 0
        lax.fori_loop(0, EN_T // 16, wrow, 0)

        def sf(j, _):
            off = j * 16
            d16 = dstb[pl.ds(off, 16)]
            for h in range(H):
                idx4[h, pl.ds(off, 16)] = d16 + h * N
            return 0
        lax.fori_loop(0, EN_T // 16, sf, 0)
        for r in range(2):
            cps = [pltpu.async_copy(nv2.at[idx4.at[2 * r + t]],
                                    bigbuf.at[pl.ds(t * EN_T, EN_T)], sem)
                   for t in range(2)]
            for c in cps:
                c.wait()

            def msg(j, _):
                jhi = (j // 16) * 16
                jlo = j - jhi
                ws = [_splat(wb[pl.ds((2 * r + t) * EN_T + jhi, 16)], jlo)
                      for t in range(2)]
                for k in range(DN // 16):
                    acc = (ws[0] * bigbuf[j, pl.ds(k * 16, 16)]
                           + ws[1] * bigbuf[EN_T + j, pl.ds(k * 16, 16)])
                    if r:
                        acc = acc + mbuf[j, pl.ds(k * 16, 16)]
                    mbuf[j, pl.ds(k * 16, 16)] = acc
                return 0
            lax.fori_loop(0, EN_T, msg, 0, unroll=4)

        def cpidx(j, _):
            off = j * 16
            sidx[pl.ds(off, 16)] = srcb[pl.ds(off, 16)]
            return 0
        lax.fori_loop(0, EN_T // 16, cpidx, 0)
        pltpu.sync_copy(mbuf, s_node.at[sidx], add=True)

        plsc.subcore_barrier()
        pltpu.sync_copy(s_node.at[pl.ds(sid * NROW_T, NROW_T)],
                        node_out.at[pl.ds(sid * NROW_T, NROW_T)])

    @pl.when(cid == 1)
    def _():
        zero_mbuf(EROW_T)
        pltpu.sync_copy(mbuf, s_edge.at[pl.ds(sid * EROW_T, EROW_T)])

        base = sid * LN_T
        pltpu.sync_copy(ls.at[pl.ds(base, LN_T)], lsb)
        pltpu.sync_copy(ld.at[pl.ds(base, LN_T)], ldb)
        pltpu.sync_copy(naf, nafb)
        plsc.subcore_barrier()

        NC_ = LN_T // CH
        def kf(j, _):
            off = j * 16
            c = j // (CH // 16)
            loff = off - c * CH
            keyb4[c, pl.ds(loff, 16)] = (lsb[pl.ds(off, 16)] * E
                                         + ldb[pl.ds(off, 16)])
            sidx4[c, pl.ds(loff, 16)] = lsb[pl.ds(off, 16)]
            return 0
        lax.fori_loop(0, LN_T // 16, kf, 0)
        cps = [pltpu.async_copy(te.at[keyb4.at[c]], winb4.at[c], sem)
               for c in range(NC_)]
        for c in cps:
            c.wait()

        def sf(j, _):
            off = j * 16
            c = j // (CH // 16)
            loff = off - c * CH
            ids = lax.iota(i32, 16) + (base + off)
            idx44[c, pl.ds(loff, 16)] = jnp.where(
                winb4[c, pl.ds(loff, 16)] == ids,
                ldb[pl.ds(off, 16)],
                jnp.full((16,), E, i32))
            return 0
        lax.fori_loop(0, LN_T // 16, sf, 0)
        for r in range(NC_ // 2):
            cps = [pltpu.async_copy(ev2.at[idx44.at[2 * r + t]],
                                    bigbuf.at[pl.ds(t * CH, CH)], sem)
                   for t in range(2)]
            for c in cps:
                c.wait()
            cps = [pltpu.async_copy(bigbuf.at[pl.ds(t * CH, CH)],
                                    s_edge.at[sidx4.at[2 * r + t]], sem,
                                    add=True)
                   for t in range(2)]
            for c in cps:
                c.wait()

        plsc.subcore_barrier()

        rbase = sid * EROW_T
        pltpu.sync_copy(s_edge.at[pl.ds(rbase, EROW_T)],
                        bigbuf.at[pl.ds(CH, EROW_T)])
        pltpu.sync_copy(gd.at[pl.ds(rbase, EROW_T)], gdb)
        def shift(j2, _):
            g16 = gdb[pl.ds(j2 * 16, 16)]
            for h in range(H):
                navb[pl.ds(h * EROW_T + j2 * 16, 16)] = plsc.load_gather(
                    nafb, [g16 + h * N])
            return 0
        lax.fori_loop(0, EROW_T // 16, shift, 0)

        def orow(i, _):
            ihi = (i // 16) * 16
            ilo = i - ihi
            acc = jnp.zeros((16,), f32)
            for h in range(H):
                nas = _splat(navb[pl.ds(h * EROW_T + ihi, 16)], ilo)
                acc = acc + nas * bigbuf[CH + i, pl.ds(h * DE, 16)]
            outb[i, :] = acc
            return 0
        lax.fori_loop(0, EROW_T, orow, 0, unroll=4)
        pltpu.sync_copy(outb, edge_out.at[pl.ds(rbase, EROW_T)])


def _agg_scratch():
    return [
            pltpu.VMEM_SHARED((N, DN), f32),
            pltpu.VMEM_SHARED((E, 128), f32),
            pltpu.VMEM((128, 128), f32),
            pltpu.VMEM((2 * CH, 128), f32),
            pltpu.VMEM((EN_T,), i32),
            pltpu.VMEM((EN_T,), i32),
            pltpu.VMEM((CH,), i32),
            pltpu.VMEM((CH,), i32),
            pltpu.VMEM((CH,), i32),
            pltpu.VMEM((H * EN_T,), f32),
            pltpu.VMEM((H * EN_T,), f32),
            pltpu.VMEM((H, CH), i32),
            pltpu.VMEM((CH,), i32),
            pltpu.VMEM((LN_T,), i32),
            pltpu.VMEM((LN_T,), i32),
            pltpu.VMEM((LN_T // CH, CH), i32),
            pltpu.VMEM((LN_T // CH, CH), i32),
            pltpu.VMEM((LN_T // CH, CH), i32),
            pltpu.VMEM((LN_T // CH, CH), i32),
            pltpu.VMEM((H * EROW_T,), f32),
            pltpu.VMEM((H * N,), f32),
            pltpu.VMEM((EROW_T,), i32),
            pltpu.VMEM((EROW_T, DE), f32),
            pltpu.SemaphoreType.DMA,
        ]


def _agg_call(gs, gd, ls, ld, tn, te, naf, eaf, nv2, ev2):
    mesh = plsc.VectorSubcoreMesh(core_axis_name="c", subcore_axis_name="s")
    fn = pl.kernel(
        _agg_body,
        out_type=(
            jax.ShapeDtypeStruct((N, DN), f32),
            jax.ShapeDtypeStruct((E, DE), f32),
        ),
        mesh=mesh,
        scratch_types=_agg_scratch(),
        compiler_params=pltpu.CompilerParams(needs_layout_passes=False),
    )
    return fn(gs, gd, ls, ld, tn, te, naf, eaf, nv2, ev2)


def kernel(node_inputs, edge_inputs, g_edge_index, lg_edge_index,
           Wnk1, bnk1, Wnk2, bnk2, Wnv1, bnv1, Wnv2, bnv2,
           Wek1, bek1, Wek2, bek2, Wev1, bev1, Wev2, bev2):
    weights = [Wnk1, bnk1.reshape(1, -1), Wnk2, bnk2.reshape(1, -1),
               Wnv1, bnv1.reshape(1, -1), Wnv2, bnv2.reshape(1, -1),
               Wek1, bek1.reshape(1, -1), Wek2, bek2.reshape(1, -1),
               Wev1, bev1.reshape(1, -1), Wev2, bev2.reshape(1, -1)]
    na2d, nv, ea2d, ev = _mlp_call(node_inputs, edge_inputs, *weights)

    gs = g_edge_index[0].astype(i32)
    gd = g_edge_index[1].astype(i32)
    ls = lg_edge_index[0].astype(i32)
    ld = lg_edge_index[1].astype(i32)

    ev2 = ev.reshape(H * E, DE)
    evcat = jnp.concatenate(
        [ev2[h * E:(h + 1) * E] for h in range(H)]
        + [jnp.zeros((E, 128 - H * DE), f32)], axis=1)
    evcat = jnp.concatenate([evcat, jnp.zeros((8, 128), f32)], axis=0)
    tn, te = _win_call(gs, gd, ls, ld)
    node_out, edge_out = _agg_call(
        gs, gd, ls, ld, tn, te,
        na2d.reshape(-1), ea2d.reshape(-1),
        nv.reshape(H * N, DN), evcat)
    return node_out, edge_out
